# baseline (speedup 1.0000x reference)
import functools

import jax
import jax.numpy as jnp
from jax import lax
from jax.experimental import pallas as pl
from jax.experimental.pallas import tpu as pltpu
from jax.experimental.pallas import tpu_sc as plsc

_F = 26
_V = 100000
_D = 32
_B = 4096

_INFO = plsc.get_sparse_core_info()
_NC = _INFO.num_cores
_NS = _INFO.num_subcores
_L = _INFO.num_lanes
_NW = _NC * _NS

_CH = 2048
_NFULL = _V // _CH
_T48 = 1664
_VT = _V - 32
_NCHUNK = _NFULL + 2
_ROWS = _F * _D
_UNITS = _F * (_D // 8)
_BV = _B // _L


def _zero16():
    return jnp.zeros((_L,), jnp.int32)


def _bucket(v):
    base = jnp.minimum(lax.shift_right_logical(v, 11), 48)
    return base + (v >= _VT).astype(jnp.int32)


def _body(tabv, tail_hbm, idx_hbm, outT, stage_v, vcol_v, off_v, vs_v, bs_v, st_v,
          ck0_v, ck1_v, ck2_v, tl_v, tp_v, blk_v, csem, wsem):
    cid = lax.axis_index("c")
    sid = lax.axis_index("s")
    wid = sid * _NC + cid
    lane = lax.iota(jnp.int32, _L)

    def sort_field(f):
        def zrow(r, c):
            off_v[r, pl.ds(0, _L)] = _zero16()
            return c
        lax.fori_loop(0, 64, zrow, 0)

        def stage_chunk(s, c):
            pltpu.sync_copy(idx_hbm.at[pl.ds(s * 512 * _F, 512 * _F)],
                            stage_v)

            def ext(k, c2):
                addrs = (k * _L + lane) * _F + f
                vcol_v[pl.ds(s * 512 + k * _L, _L)] = plsc.load_gather(
                    stage_v, [addrs])
                return c2

            lax.fori_loop(0, 512 // _L, ext, 0)
            return c

        lax.fori_loop(0, _B // 512, stage_chunk, 0)

        def hist(k, c):
            v = vcol_v[pl.ds(k * _L, _L)]
            cv = _bucket(v)
            plsc.addupdate_scatter(off_v, [cv, lane], _zero16() + 1)
            return c

        lax.fori_loop(0, _BV, hist, 0)

        def scan_row(r, carry):
            cnt = off_v[r, pl.ds(0, _L)]
            inc = plsc.cumsum(cnt)
            exc = inc - cnt + carry
            off_v[r, pl.ds(0, _L)] = exc
            last = inc[_L - 1]
            return carry + last

        lax.fori_loop(0, 64, scan_row, jnp.int32(0))

        def starts(q, c):
            cv = q * _L + lane
            st_v[pl.ds(q * _L, _L)] = plsc.load_gather(off_v, [cv, _zero16()])
            return c

        lax.fori_loop(0, 64 // _L, starts, 0)

        def place(k, c):
            v = vcol_v[pl.ds(k * _L, _L)]
            cv = _bucket(v)
            pos = plsc.load_gather(off_v, [cv, lane])
            plsc.store_scatter(vs_v, [pos], v)
            plsc.store_scatter(bs_v, [pos], k * _L + lane)
            plsc.addupdate_scatter(off_v, [cv, lane], _zero16() + 1)
            return c

        lax.fori_loop(0, _BV, place, 0)

    def do_unit(u):
        f = lax.div(u, 4)
        g = lax.rem(u, 4)
        row0 = f * _D + g * 8

        sort_field(f)

        def start_full(c, buf):
            pltpu.make_async_copy(
                tabv.at[pl.ds(row0, 8), pl.ds(c * _CH, _CH)], buf,
                csem).start()

        def wait_full(c, buf):
            pltpu.make_async_copy(
                tabv.at[pl.ds(row0, 8), pl.ds(c * _CH, _CH)], buf,
                csem).wait()

        def extract(c, lo, sz, buf):
            svec = plsc.load_gather(st_v, [_zero16() + c + lane])
            s_c = svec[0]
            e_c = svec[1]

            def ext_block(i, cc):
                base = i * _L
                v = vs_v[pl.ds(base, _L)]
                b = bs_v[pl.ds(base, _L)]
                inb = jnp.logical_and(v >= lo, v < lo + sz)
                col = jnp.clip(v - lo, 0, sz - 1)
                for sub in range(8):
                    vals = plsc.load_gather(buf, [_zero16() + sub, col],
                                            mask=inb)
                    plsc.store_scatter(blk_v, [_zero16() + sub, b], vals,
                                       mask=inb)
                return cc

            lax.fori_loop(lax.div(s_c, _L), lax.div(e_c + _L - 1, _L),
                          ext_block, 0)

        start_full(0, ck0_v)
        start_full(1, ck1_v)
        pltpu.make_async_copy(
            tabv.at[pl.ds(row0, 8), pl.ds(_NFULL * _CH, _T48)], tl_v,
            wsem).start()
        pltpu.make_async_copy(
            tail_hbm.at[pl.ds(row0, 8), pl.ds(0, 128)], tp_v, wsem).start()

        def triple(j, c):
            c0 = 3 * j
            start_full(c0 + 2, ck2_v)
            wait_full(c0, ck0_v)
            extract(c0, c0 * _CH, _CH, ck0_v)

            @pl.when(c0 + 3 < _NFULL)
            def _():
                start_full(c0 + 3, ck0_v)

            wait_full(c0 + 1, ck1_v)
            extract(c0 + 1, (c0 + 1) * _CH, _CH, ck1_v)

            @pl.when(c0 + 4 < _NFULL)
            def _():
                start_full(c0 + 4, ck1_v)

            wait_full(c0 + 2, ck2_v)
            extract(c0 + 2, (c0 + 2) * _CH, _CH, ck2_v)
            return c

        lax.fori_loop(0, _NFULL // 3, triple, 0)

        pltpu.make_async_copy(
            tabv.at[pl.ds(row0, 8), pl.ds(_NFULL * _CH, _T48)], tl_v,
            wsem).wait()
        extract(_NFULL, _NFULL * _CH, _T48, tl_v)
        pltpu.make_async_copy(
            tail_hbm.at[pl.ds(row0, 8), pl.ds(0, 128)], tp_v, wsem).wait()
        extract(_NFULL + 1, _VT, 32, tp_v)

        pltpu.make_async_copy(
            blk_v, outT.at[f, pl.ds(g * 8, 8), pl.ds(0, _B)], wsem,
        ).start()
        pltpu.make_async_copy(
            blk_v, outT.at[f, pl.ds(g * 8, 8), pl.ds(0, _B)], wsem,
        ).wait()

    def unit_loop(i, c):
        u = wid + 32 * i

        @pl.when(u < _UNITS)
        def _():
            do_unit(u)

        return c

    lax.fori_loop(0, 4, unit_loop, 0)


_sc_sweep = functools.partial(
    pl.kernel,
    mesh=plsc.VectorSubcoreMesh(core_axis_name="c", subcore_axis_name="s"),
    compiler_params=pltpu.CompilerParams(
        use_tc_tiling_on_sc=True, needs_layout_passes=False),
    out_type=jax.ShapeDtypeStruct((_F, _D, _B), jnp.float32),
    scratch_types=[
        pltpu.VMEM((512 * _F,), jnp.int32),
        pltpu.VMEM((_B,), jnp.int32),
        pltpu.VMEM((64, _L), jnp.int32),
        pltpu.VMEM((_B,), jnp.int32),
        pltpu.VMEM((_B,), jnp.int32),
        pltpu.VMEM((80,), jnp.int32),
        pltpu.VMEM((8, _CH), jnp.float32),
        pltpu.VMEM((8, _CH), jnp.float32),
        pltpu.VMEM((8, _CH), jnp.float32),
        pltpu.VMEM((8, _T48), jnp.float32),
        pltpu.VMEM((8, 128), jnp.float32),
        pltpu.VMEM((8, _B), jnp.float32),
        pltpu.SemaphoreType.DMA,
        pltpu.SemaphoreType.DMA,
    ],
)(_body)


@jax.jit
def kernel(indices, tables):
    tabv = jnp.transpose(tables, (0, 2, 1)).reshape(_ROWS, _V)
    tail = jnp.pad(jnp.transpose(tables[:, _VT:, :], (0, 2, 1)),
                   ((0, 0), (0, 0), (0, 96))).reshape(_ROWS, 128)
    outT = _sc_sweep(tabv, tail, indices.reshape(_B * _F))
    return jnp.transpose(outT, (2, 0, 1))


# speedup vs baseline: 1.2534x; 1.2534x over previous
import functools

import jax
import jax.numpy as jnp
from jax import lax
from jax.experimental import pallas as pl
from jax.experimental.pallas import tpu as pltpu
from jax.experimental.pallas import tpu_sc as plsc

_F = 26
_V = 100000
_D = 32
_B = 4096

_INFO = plsc.get_sparse_core_info()
_NC = _INFO.num_cores
_NS = _INFO.num_subcores
_L = _INFO.num_lanes
_NW = _NC * _NS

_CH = 2048
_NFULL = _V // _CH
_T48 = 1664
_VT = _V - 32
_NCHUNK = _NFULL + 2
_ROWS = _F * _D
_UNITS = _F * (_D // 8)
_BV = _B // _L


def _zero16():
    return jnp.zeros((_L,), jnp.int32)


def _bucket(v):
    base = jnp.minimum(lax.shift_right_logical(v, 11), 48)
    return base + (v >= _VT).astype(jnp.int32)


def _body(tabv, tail_hbm, idx_hbm, outT, vcol_v, off_v, vs_v, bs_v, st_v,
          ck0_v, ck1_v, ck2_v, tl_v, tp_v, blk_v, csem, wsem):
    cid = lax.axis_index("c")
    sid = lax.axis_index("s")
    wid = sid * _NC + cid
    lane = lax.iota(jnp.int32, _L)

    def sort_field(f):
        def zrow(r, c):
            off_v[r, pl.ds(0, _L)] = _zero16()
            return c
        lax.fori_loop(0, 64, zrow, 0)

        pltpu.sync_copy(idx_hbm.at[pl.ds(f * _B, _B)], vcol_v)

        def hist(k, c):
            v = vcol_v[pl.ds(k * _L, _L)]
            cv = _bucket(v)
            plsc.addupdate_scatter(off_v, [cv, lane], _zero16() + 1)
            return c

        lax.fori_loop(0, _BV, hist, 0)

        def scan_row(r, carry):
            cnt = off_v[r, pl.ds(0, _L)]
            inc = plsc.cumsum(cnt)
            exc = inc - cnt + carry
            off_v[r, pl.ds(0, _L)] = exc
            last = inc[_L - 1]
            return carry + last

        lax.fori_loop(0, 64, scan_row, jnp.int32(0))

        def starts(q, c):
            cv = q * _L + lane
            st_v[pl.ds(q * _L, _L)] = plsc.load_gather(off_v, [cv, _zero16()])
            return c

        lax.fori_loop(0, 64 // _L, starts, 0)

        def place(k, c):
            v = vcol_v[pl.ds(k * _L, _L)]
            cv = _bucket(v)
            pos = plsc.load_gather(off_v, [cv, lane])
            plsc.store_scatter(vs_v, [pos], v)
            plsc.store_scatter(bs_v, [pos], k * _L + lane)
            plsc.addupdate_scatter(off_v, [cv, lane], _zero16() + 1)
            return c

        lax.fori_loop(0, _BV, place, 0)

    def do_unit(u):
        f = lax.div(u, 4)
        g = lax.rem(u, 4)
        row0 = f * _D + g * 8

        sort_field(f)

        def start_full(c, buf):
            pltpu.make_async_copy(
                tabv.at[pl.ds(row0, 8), pl.ds(c * _CH, _CH)], buf,
                csem).start()

        def wait_full(c, buf):
            pltpu.make_async_copy(
                tabv.at[pl.ds(row0, 8), pl.ds(c * _CH, _CH)], buf,
                csem).wait()

        def extract(c, lo, sz, buf):
            svec = plsc.load_gather(st_v, [_zero16() + c + lane])
            s_c = svec[0]
            e_c = svec[1]

            def ext_block(i, cc):
                base = i * _L
                v = vs_v[pl.ds(base, _L)]
                b = bs_v[pl.ds(base, _L)]
                inb = jnp.logical_and(v >= lo, v < lo + sz)
                col = jnp.clip(v - lo, 0, sz - 1)
                for sub in range(8):
                    vals = plsc.load_gather(buf, [_zero16() + sub, col],
                                            mask=inb)
                    plsc.store_scatter(blk_v, [_zero16() + sub, b], vals,
                                       mask=inb)
                return cc

            lax.fori_loop(lax.div(s_c, _L), lax.div(e_c + _L - 1, _L),
                          ext_block, 0)

        start_full(0, ck0_v)
        start_full(1, ck1_v)
        pltpu.make_async_copy(
            tabv.at[pl.ds(row0, 8), pl.ds(_NFULL * _CH, _T48)], tl_v,
            wsem).start()
        pltpu.make_async_copy(
            tail_hbm.at[pl.ds(row0, 8), pl.ds(0, 128)], tp_v, wsem).start()

        def triple(j, c):
            c0 = 3 * j
            start_full(c0 + 2, ck2_v)
            wait_full(c0, ck0_v)
            extract(c0, c0 * _CH, _CH, ck0_v)

            @pl.when(c0 + 3 < _NFULL)
            def _():
                start_full(c0 + 3, ck0_v)

            wait_full(c0 + 1, ck1_v)
            extract(c0 + 1, (c0 + 1) * _CH, _CH, ck1_v)

            @pl.when(c0 + 4 < _NFULL)
            def _():
                start_full(c0 + 4, ck1_v)

            wait_full(c0 + 2, ck2_v)
            extract(c0 + 2, (c0 + 2) * _CH, _CH, ck2_v)
            return c

        lax.fori_loop(0, _NFULL // 3, triple, 0)

        pltpu.make_async_copy(
            tabv.at[pl.ds(row0, 8), pl.ds(_NFULL * _CH, _T48)], tl_v,
            wsem).wait()
        extract(_NFULL, _NFULL * _CH, _T48, tl_v)
        pltpu.make_async_copy(
            tail_hbm.at[pl.ds(row0, 8), pl.ds(0, 128)], tp_v, wsem).wait()
        extract(_NFULL + 1, _VT, 32, tp_v)

        pltpu.make_async_copy(
            blk_v, outT.at[f, pl.ds(g * 8, 8), pl.ds(0, _B)], wsem,
        ).start()
        pltpu.make_async_copy(
            blk_v, outT.at[f, pl.ds(g * 8, 8), pl.ds(0, _B)], wsem,
        ).wait()

    def unit_loop(i, c):
        u = wid + 32 * i

        @pl.when(u < _UNITS)
        def _():
            do_unit(u)

        return c

    lax.fori_loop(0, 4, unit_loop, 0)


_sc_sweep = functools.partial(
    pl.kernel,
    mesh=plsc.VectorSubcoreMesh(core_axis_name="c", subcore_axis_name="s"),
    compiler_params=pltpu.CompilerParams(
        use_tc_tiling_on_sc=True, needs_layout_passes=False),
    out_type=jax.ShapeDtypeStruct((_F, _D, _B), jnp.float32),
    scratch_types=[
        pltpu.VMEM((_B,), jnp.int32),
        pltpu.VMEM((64, _L), jnp.int32),
        pltpu.VMEM((_B,), jnp.int32),
        pltpu.VMEM((_B,), jnp.int32),
        pltpu.VMEM((80,), jnp.int32),
        pltpu.VMEM((8, _CH), jnp.float32),
        pltpu.VMEM((8, _CH), jnp.float32),
        pltpu.VMEM((8, _CH), jnp.float32),
        pltpu.VMEM((8, _T48), jnp.float32),
        pltpu.VMEM((8, 128), jnp.float32),
        pltpu.VMEM((8, _B), jnp.float32),
        pltpu.SemaphoreType.DMA,
        pltpu.SemaphoreType.DMA,
    ],
)(_body)


@jax.jit
def kernel(indices, tables):
    tabv = jnp.transpose(tables, (0, 2, 1)).reshape(_ROWS, _V)
    tail = jnp.pad(jnp.transpose(tables[:, _VT:, :], (0, 2, 1)),
                   ((0, 0), (0, 0), (0, 96))).reshape(_ROWS, 128)
    outT = _sc_sweep(tabv, tail, jnp.transpose(indices).reshape(_F * _B))
    return jnp.transpose(outT, (2, 0, 1))
